# baseline (speedup 1.0000x reference)
import functools

import jax
import jax.numpy as jnp
from jax import lax
from jax.experimental import pallas as pl
from jax.experimental.pallas import tpu as pltpu
from jax.experimental.pallas import tpu_sc as plsc

_EPS = 1e-09
_THRESHOLD = 0.2
_CAPACITY_FACTOR = 1.25
_MIN_EXPERT = 4
_CHUNK = 256


def _excl_cumsum_tokens(m, n, e):
    c = _CHUNK
    nc = n // c
    ri = lax.broadcasted_iota(jnp.int32, (c, c), 0)
    ci = lax.broadcasted_iota(jnp.int32, (c, c), 1)
    ltri = (ci < ri).astype(jnp.float32)
    parts = []
    run = jnp.zeros((1, e), jnp.float32)
    for k in range(nc):
        chunk = m[k * c:(k + 1) * c, :]
        within = jnp.dot(ltri, chunk, preferred_element_type=jnp.float32)
        parts.append(within + run)
        run = run + jnp.sum(chunk, axis=0, keepdims=True)
    return jnp.concatenate(parts, axis=0)


def _route_body(n, e, cap, x_ref, w_ref, off1_ref, val1_ref, off2_ref,
                val2_ref):
    xb = x_ref[0]
    w = w_ref[...]
    logits = jnp.dot(xb.astype(jnp.bfloat16), w.astype(jnp.bfloat16),
                     preferred_element_type=jnp.float32)
    lmax = jnp.max(logits, axis=1, keepdims=True)
    ex = jnp.exp(logits - lmax)
    g = ex / jnp.sum(ex, axis=1, keepdims=True)

    eio = lax.broadcasted_iota(jnp.int32, (n, e), 1)
    m1 = jnp.max(g, axis=1, keepdims=True)
    idx1 = jnp.min(jnp.where(g == m1, eio, e), axis=1, keepdims=True)
    mask1 = (eio == idx1).astype(jnp.float32)
    gm = g * (1.0 - mask1)
    m2 = jnp.max(gm, axis=1, keepdims=True)
    idx2 = jnp.min(jnp.where(gm == m2, eio, e), axis=1, keepdims=True)

    g1n = m1 / (m1 + m2 + _EPS)
    g2n = m2 / (g1n + m2 + _EPS)
    thr = (g2n > _THRESHOLD).astype(jnp.float32)
    mask2 = (eio == idx2).astype(jnp.float32) * thr

    p1_full = _excl_cumsum_tokens(mask1, n, e)
    q2_full = _excl_cumsum_tokens(mask2, n, e)
    p1 = jnp.sum(p1_full * mask1, axis=1, keepdims=True)
    cnt1 = jnp.minimum(jnp.sum(mask1, axis=0, keepdims=True),
                       float(cap))
    p2 = jnp.sum((q2_full + cnt1) * mask2, axis=1, keepdims=True)

    keep1 = (p1 < float(cap)).astype(jnp.float32)
    keep2 = thr * (p2 < float(cap)).astype(jnp.float32)
    fcap = float(cap - 1)
    trash = e * cap + (lax.broadcasted_iota(jnp.int32, (n, 1), 0) & 15)
    s1 = idx1 * cap + jnp.minimum(p1, fcap).astype(jnp.int32)
    s2 = idx2 * cap + jnp.minimum(p2, fcap).astype(jnp.int32)
    off1_ref[0] = jnp.where(keep1 > 0.0, s1, trash)
    off2_ref[0] = jnp.where(keep2 > 0.0, s2, trash)
    val1_ref[0] = g1n * keep1
    val2_ref[0] = g2n * keep2


def _route(x, w):
    b, n, d = x.shape
    e = w.shape[-1]
    cap = max(min(n, int(n * _CAPACITY_FACTOR / e)), _MIN_EXPERT)
    out_sd = [
        jax.ShapeDtypeStruct((b, n, 1), jnp.int32),
        jax.ShapeDtypeStruct((b, n, 1), jnp.float32),
        jax.ShapeDtypeStruct((b, n, 1), jnp.int32),
        jax.ShapeDtypeStruct((b, n, 1), jnp.float32),
    ]
    osp = pl.BlockSpec((1, n, 1), lambda i: (i, 0, 0))
    return pl.pallas_call(
        functools.partial(_route_body, n, e, cap),
        grid=(b,),
        in_specs=[
            pl.BlockSpec((1, n, d), lambda i: (i, 0, 0)),
            pl.BlockSpec((d, e), lambda i: (0, 0)),
        ],
        out_specs=[osp, osp, osp, osp],
        out_shape=out_sd,
    )(x, w)


def _tables(off1, val1, off2, val2, b, n, ecap):
    info = plsc.get_sparse_core_info()
    nc = info.num_cores
    mesh = plsc.VectorSubcoreMesh(core_axis_name="c", subcore_axis_name="s")

    def body(off1_h, val1_h, off2_h, val2_h, tok_h, gate_h, off1_v, val1_v,
             off2_v, val2_v, tok_v, gate_v):
        wid = lax.axis_index("s") * nc + lax.axis_index("c")

        @pl.when(wid < b)
        def _():
            base = wid * n
            pltpu.sync_copy(off1_h.at[pl.ds(base, n)], off1_v)
            pltpu.sync_copy(val1_h.at[pl.ds(base, n)], val1_v)
            pltpu.sync_copy(off2_h.at[pl.ds(base, n)], off2_v)
            pltpu.sync_copy(val2_h.at[pl.ds(base, n)], val2_v)

            def zero_body(i, carry):
                sl = pl.ds(i * 16, 16)
                tok_v[sl] = jnp.zeros((16,), jnp.int32)
                gate_v[sl] = jnp.zeros((16,), jnp.float32)
                return carry

            lax.fori_loop(0, ecap // 16, zero_body, 0)

            lane = jnp.arange(16, dtype=jnp.int32)

            def scat_body(i, carry):
                sl = pl.ds(i * 16, 16)
                t = lane + i * 16
                o1 = off1_v[sl]
                plsc.store_scatter(tok_v, [o1], t)
                plsc.store_scatter(gate_v, [o1], val1_v[sl])
                o2 = off2_v[sl]
                plsc.store_scatter(tok_v, [o2], t)
                plsc.store_scatter(gate_v, [o2], val2_v[sl])
                return carry

            lax.fori_loop(0, n // 16, scat_body, 0)
            pltpu.sync_copy(tok_v.at[pl.ds(0, ecap)], tok_h.at[wid])
            pltpu.sync_copy(gate_v.at[pl.ds(0, ecap)], gate_h.at[wid])

    run = pl.kernel(
        body,
        out_type=(
            jax.ShapeDtypeStruct((b, ecap), jnp.int32),
            jax.ShapeDtypeStruct((b, ecap), jnp.float32),
        ),
        mesh=mesh,
        compiler_params=pltpu.CompilerParams(needs_layout_passes=False),
        scratch_types=[
            pltpu.VMEM((n,), jnp.int32),
            pltpu.VMEM((n,), jnp.float32),
            pltpu.VMEM((n,), jnp.int32),
            pltpu.VMEM((n,), jnp.float32),
            pltpu.VMEM((ecap + 16,), jnp.int32),
            pltpu.VMEM((ecap + 16,), jnp.float32),
        ],
    )
    return run(off1, val1, off2, val2)


def _expand_sc(tok_flat, gate_flat, b, n, e, cap):
    info = plsc.get_sparse_core_info()
    nc = info.num_cores
    mesh = plsc.VectorSubcoreMesh(core_axis_name="c", subcore_axis_name="s")
    rows = b * e * cap

    def body(tok_h, gate_h, out_h, tok_v, gate_v, buf):
        wid = lax.axis_index("s") * nc + lax.axis_index("c")

        @pl.when(wid < b * e)
        def _():
            pltpu.sync_copy(tok_h.at[pl.ds(wid * cap, cap)], tok_v)
            pltpu.sync_copy(gate_h.at[pl.ds(wid * cap, cap)], gate_v)

            zrow = jnp.zeros((16,), jnp.float32)
            nvec = n // 16

            def zero_body(i, carry):
                r = i // nvec
                col = (i - r * nvec) * 16
                buf[r, pl.ds(col, 16)] = zrow
                return carry

            lax.fori_loop(0, 17 * nvec, zero_body, 0)

            lane = jnp.arange(16, dtype=jnp.int32)

            def chunk_body(c, carry):
                t = tok_v[pl.ds(c * 16, 16)]
                gv = gate_v[pl.ds(c * 16, 16)]
                keep = gv > 0.0
                row = jnp.where(keep, lane, 16)
                col = jnp.where(keep, t, lane)
                plsc.store_scatter(buf, [row, col], gv)
                pltpu.sync_copy(
                    buf.at[pl.ds(0, 16)],
                    out_h.at[pl.ds(wid * cap + c * 16, 16)])
                plsc.store_scatter(buf, [row, col],
                                   jnp.zeros((16,), jnp.float32))
                return carry

            lax.fori_loop(0, cap // 16, chunk_body, 0)

    run = pl.kernel(
        body,
        out_type=jax.ShapeDtypeStruct((rows, n), jnp.float32),
        mesh=mesh,
        compiler_params=pltpu.CompilerParams(needs_layout_passes=False),
        scratch_types=[
            pltpu.VMEM((cap,), jnp.int32),
            pltpu.VMEM((cap,), jnp.float32),
            pltpu.VMEM((17, n), jnp.float32),
        ],
    )
    return run(tok_flat, gate_flat)


def _expand_body(tspan, tok_ref, gate_ref, out_ref):
    t0 = pl.program_id(1) * tspan
    tok = tok_ref[0] - t0
    gate = gate_ref[0]
    shape = tok.shape + (tspan,)
    tio = lax.broadcasted_iota(jnp.int32, shape, 2)
    out_ref[0] = jnp.where(tio == tok[..., None], gate[..., None], 0.0)


def _expand(tok, gate, b, n, e, cap, tspan=1024):
    out = pl.pallas_call(
        functools.partial(_expand_body, tspan),
        grid=(b, n // tspan),
        in_specs=[
            pl.BlockSpec((1, e, cap), lambda i, j: (i, 0, 0)),
            pl.BlockSpec((1, e, cap), lambda i, j: (i, 0, 0)),
        ],
        out_specs=pl.BlockSpec((1, e, cap, tspan), lambda i, j: (i, 0, 0, j)),
        out_shape=jax.ShapeDtypeStruct((b, e, cap, n), jnp.float32),
    )(tok, gate)
    return jnp.transpose(out, (0, 3, 1, 2))


def kernel(x, gating_weights):
    b, n, d = x.shape
    e = gating_weights.shape[-1]
    cap = max(min(n, int(n * _CAPACITY_FACTOR / e)), _MIN_EXPERT)
    off1, val1, off2, val2 = _route(x, gating_weights)
    tok, gate = _tables(off1.reshape(b * n), val1.reshape(b * n),
                        off2.reshape(b * n), val2.reshape(b * n),
                        b=b, n=n, ecap=e * cap)
    out = _expand_sc(tok.reshape(b * e * cap), gate.reshape(b * e * cap),
                     b=b, n=n, e=e, cap=cap)
    return jnp.transpose(out.reshape(b, e, cap, n), (0, 3, 1, 2))


# speedup vs baseline: 1.0875x; 1.0875x over previous
import functools

import jax
import jax.numpy as jnp
from jax import lax
from jax.experimental import pallas as pl
from jax.experimental.pallas import tpu as pltpu
from jax.experimental.pallas import tpu_sc as plsc

_EPS = 1e-09
_THRESHOLD = 0.2
_CAPACITY_FACTOR = 1.25
_MIN_EXPERT = 4
_CHUNK = 256


def _excl_cumsum_tokens(m, n, e):
    c = _CHUNK
    nc = n // c
    ri = lax.broadcasted_iota(jnp.int32, (c, c), 0)
    ci = lax.broadcasted_iota(jnp.int32, (c, c), 1)
    ltri = (ci < ri).astype(jnp.float32)
    parts = []
    run = jnp.zeros((1, e), jnp.float32)
    for k in range(nc):
        chunk = m[k * c:(k + 1) * c, :]
        within = jnp.dot(ltri, chunk, preferred_element_type=jnp.float32)
        parts.append(within + run)
        run = run + jnp.sum(chunk, axis=0, keepdims=True)
    return jnp.concatenate(parts, axis=0)


def _route_body(n, e, cap, x_ref, w_ref, off1_ref, val1_ref, off2_ref,
                val2_ref):
    xb = x_ref[0]
    w = w_ref[...]
    logits = jnp.dot(xb.astype(jnp.bfloat16), w.astype(jnp.bfloat16),
                     preferred_element_type=jnp.float32)
    lmax = jnp.max(logits, axis=1, keepdims=True)
    ex = jnp.exp(logits - lmax)
    g = ex / jnp.sum(ex, axis=1, keepdims=True)

    eio = lax.broadcasted_iota(jnp.int32, (n, e), 1)
    m1 = jnp.max(g, axis=1, keepdims=True)
    idx1 = jnp.min(jnp.where(g == m1, eio, e), axis=1, keepdims=True)
    mask1 = (eio == idx1).astype(jnp.float32)
    gm = g * (1.0 - mask1)
    m2 = jnp.max(gm, axis=1, keepdims=True)
    idx2 = jnp.min(jnp.where(gm == m2, eio, e), axis=1, keepdims=True)

    g1n = m1 / (m1 + m2 + _EPS)
    g2n = m2 / (g1n + m2 + _EPS)
    thr = (g2n > _THRESHOLD).astype(jnp.float32)
    mask2 = (eio == idx2).astype(jnp.float32) * thr

    p1_full = _excl_cumsum_tokens(mask1, n, e)
    q2_full = _excl_cumsum_tokens(mask2, n, e)
    p1 = jnp.sum(p1_full * mask1, axis=1, keepdims=True)
    cnt1 = jnp.minimum(jnp.sum(mask1, axis=0, keepdims=True),
                       float(cap))
    p2 = jnp.sum((q2_full + cnt1) * mask2, axis=1, keepdims=True)

    keep1 = (p1 < float(cap)).astype(jnp.float32)
    keep2 = thr * (p2 < float(cap)).astype(jnp.float32)
    fcap = float(cap - 1)
    trash = e * cap + (lax.broadcasted_iota(jnp.int32, (n, 1), 0) & 15)
    s1 = idx1 * cap + jnp.minimum(p1, fcap).astype(jnp.int32)
    s2 = idx2 * cap + jnp.minimum(p2, fcap).astype(jnp.int32)
    off1_ref[0] = jnp.where(keep1 > 0.0, s1, trash)
    off2_ref[0] = jnp.where(keep2 > 0.0, s2, trash)
    val1_ref[0] = g1n * keep1
    val2_ref[0] = g2n * keep2


def _route(x, w):
    b, n, d = x.shape
    e = w.shape[-1]
    cap = max(min(n, int(n * _CAPACITY_FACTOR / e)), _MIN_EXPERT)
    out_sd = [
        jax.ShapeDtypeStruct((b, n, 1), jnp.int32),
        jax.ShapeDtypeStruct((b, n, 1), jnp.float32),
        jax.ShapeDtypeStruct((b, n, 1), jnp.int32),
        jax.ShapeDtypeStruct((b, n, 1), jnp.float32),
    ]
    osp = pl.BlockSpec((1, n, 1), lambda i: (i, 0, 0))
    return pl.pallas_call(
        functools.partial(_route_body, n, e, cap),
        grid=(b,),
        in_specs=[
            pl.BlockSpec((1, n, d), lambda i: (i, 0, 0)),
            pl.BlockSpec((d, e), lambda i: (0, 0)),
        ],
        out_specs=[osp, osp, osp, osp],
        out_shape=out_sd,
    )(x, w)


def _tables(off1, val1, off2, val2, b, n, ecap):
    info = plsc.get_sparse_core_info()
    nc = info.num_cores
    mesh = plsc.VectorSubcoreMesh(core_axis_name="c", subcore_axis_name="s")

    def body(off1_h, val1_h, off2_h, val2_h, tok_h, gate_h, off1_v, val1_v,
             off2_v, val2_v, tok_v, gate_v):
        wid = lax.axis_index("s") * nc + lax.axis_index("c")

        @pl.when(wid < b)
        def _():
            base = wid * n
            pltpu.sync_copy(off1_h.at[pl.ds(base, n)], off1_v)
            pltpu.sync_copy(val1_h.at[pl.ds(base, n)], val1_v)
            pltpu.sync_copy(off2_h.at[pl.ds(base, n)], off2_v)
            pltpu.sync_copy(val2_h.at[pl.ds(base, n)], val2_v)

            def zero_body(i, carry):
                sl = pl.ds(i * 16, 16)
                tok_v[sl] = jnp.zeros((16,), jnp.int32)
                gate_v[sl] = jnp.zeros((16,), jnp.float32)
                return carry

            lax.fori_loop(0, ecap // 16, zero_body, 0)

            lane = jnp.arange(16, dtype=jnp.int32)

            def scat_body(i, carry):
                sl = pl.ds(i * 16, 16)
                t = lane + i * 16
                o1 = off1_v[sl]
                plsc.store_scatter(tok_v, [o1], t)
                plsc.store_scatter(gate_v, [o1], val1_v[sl])
                o2 = off2_v[sl]
                plsc.store_scatter(tok_v, [o2], t)
                plsc.store_scatter(gate_v, [o2], val2_v[sl])
                return carry

            lax.fori_loop(0, n // 16, scat_body, 0)
            pltpu.sync_copy(tok_v.at[pl.ds(0, ecap)], tok_h.at[wid])
            pltpu.sync_copy(gate_v.at[pl.ds(0, ecap)], gate_h.at[wid])

    run = pl.kernel(
        body,
        out_type=(
            jax.ShapeDtypeStruct((b, ecap), jnp.int32),
            jax.ShapeDtypeStruct((b, ecap), jnp.float32),
        ),
        mesh=mesh,
        compiler_params=pltpu.CompilerParams(needs_layout_passes=False),
        scratch_types=[
            pltpu.VMEM((n,), jnp.int32),
            pltpu.VMEM((n,), jnp.float32),
            pltpu.VMEM((n,), jnp.int32),
            pltpu.VMEM((n,), jnp.float32),
            pltpu.VMEM((ecap + 16,), jnp.int32),
            pltpu.VMEM((ecap + 16,), jnp.float32),
        ],
    )
    return run(off1, val1, off2, val2)


def _expand_body(tspan, tok_ref, gate_ref, out_ref):
    t0 = pl.program_id(1) * tspan
    tok = tok_ref[0] - t0
    gate = gate_ref[0]
    shape = tok.shape + (tspan,)
    tio = lax.broadcasted_iota(jnp.int32, shape, 2)
    out_ref[0] = jnp.where(tio == tok[..., None], gate[..., None], 0.0)


def _expand(tok, gate, b, n, e, cap, tspan=1024):
    out = pl.pallas_call(
        functools.partial(_expand_body, tspan),
        grid=(b, n // tspan),
        in_specs=[
            pl.BlockSpec((1, e, cap), lambda i, j: (i, 0, 0)),
            pl.BlockSpec((1, e, cap), lambda i, j: (i, 0, 0)),
        ],
        out_specs=pl.BlockSpec((1, e, cap, tspan), lambda i, j: (i, 0, 0, j)),
        out_shape=jax.ShapeDtypeStruct((b, e, cap, n), jnp.float32),
    )(tok, gate)
    return jnp.transpose(out, (0, 3, 1, 2))


def kernel(x, gating_weights):
    b, n, d = x.shape
    e = gating_weights.shape[-1]
    cap = max(min(n, int(n * _CAPACITY_FACTOR / e)), _MIN_EXPERT)
    off1, val1, off2, val2 = _route(x, gating_weights)
    tok, gate = _tables(off1.reshape(b * n), val1.reshape(b * n),
                        off2.reshape(b * n), val2.reshape(b * n),
                        b=b, n=n, ecap=e * cap)
    return _expand(tok.reshape(b, e, cap), gate.reshape(b, e, cap),
                   b=b, n=n, e=e, cap=cap)
